# baseline (speedup 1.0000x reference)
import functools

import jax
import jax.numpy as jnp
from jax import lax
from jax.experimental import pallas as pl
from jax.experimental.pallas import tpu as pltpu
from jax.experimental.pallas import tpu_sc as plsc

N = 10000
D_IN = 256
Z_DIM = 256
H = 128
NP = 10240
E = 320000
CW = 128
NS = 16
NC = 2
CH = 160
EPAD = NS * CH * CW
ROWS_PER_TILE = NP // NS
BLK = 1024


def _mm_norm_body(x_ref, w_ref, b_ref, h_ref):
    h = lax.dot_general(x_ref[...], w_ref[...],
                        dimension_numbers=(((1,), (1,)), ((), ())),
                        preferred_element_type=jnp.float32)
    h = h + b_ref[...]
    l2 = jnp.sqrt(jnp.sum(h * h, axis=1, keepdims=True))
    h_ref[...] = h / jnp.maximum(l2, 1e-12) * 1.8


def _mm_norm(xp, W, b):
    return pl.pallas_call(
        _mm_norm_body,
        grid=(NP // BLK,),
        in_specs=[
            pl.BlockSpec((BLK, D_IN), lambda i: (i, 0)),
            pl.BlockSpec((Z_DIM, D_IN), lambda i: (0, 0)),
            pl.BlockSpec((1, Z_DIM), lambda i: (0, 0)),
        ],
        out_specs=pl.BlockSpec((BLK, Z_DIM), lambda i: (i, 0)),
        out_shape=jax.ShapeDtypeStruct((NP, Z_DIM), jnp.float32),
    )(xp, W, b.reshape(1, Z_DIM))


def _deg_body(dsti, out, dstv, onesb, zb, dacc):
    c = lax.axis_index("c")
    s = lax.axis_index("s")
    pltpu.sync_copy(dsti.at[s, pl.ds((CH // 2) * c, CH // 2)], dstv)
    for j in range(8):
        onesb[pl.ds(16 * j, 16)] = jnp.ones((16,), jnp.float32)

    @pl.loop(0, ROWS_PER_TILE, step=16)
    def _(r):
        zb[pl.ds(r, 16)] = jnp.zeros((16,), jnp.float32)

    pltpu.sync_copy(zb, dacc.at[pl.ds(s * ROWS_PER_TILE, ROWS_PER_TILE)])
    plsc.subcore_barrier()

    @pl.loop(0, CH // 2)
    def _(ch):
        pltpu.sync_copy(onesb, dacc.at[dstv.at[ch]], add=True)

    plsc.subcore_barrier()
    pltpu.sync_copy(dacc.at[pl.ds(s * ROWS_PER_TILE, ROWS_PER_TILE)],
                    out.at[pl.ds(c * NP + s * ROWS_PER_TILE, ROWS_PER_TILE)])


def _deg_counts(dsti):
    mesh = plsc.VectorSubcoreMesh(core_axis_name="c", subcore_axis_name="s")
    k = pl.kernel(
        _deg_body,
        out_type=jax.ShapeDtypeStruct((NC * NP,), jnp.float32),
        mesh=mesh,
        scratch_types=[
            pltpu.VMEM((CH // 2, CW), jnp.int32),
            pltpu.VMEM((CW,), jnp.float32),
            pltpu.VMEM((ROWS_PER_TILE,), jnp.float32),
            pltpu.VMEM_SHARED((NP,), jnp.float32),
        ],
    )
    return k(dsti)


def _scale_body(h_ref, dp_ref, g_ref):
    cnt = dp_ref[0] + dp_ref[1] + 1.0
    dinv = lax.rsqrt(cnt)
    g_ref[...] = h_ref[...] * dinv


def _scale(h, dparts):
    return pl.pallas_call(
        _scale_body,
        grid=(NP // BLK, 2),
        in_specs=[
            pl.BlockSpec((BLK, H), lambda i, j: (i, j)),
            pl.BlockSpec((2, BLK, 1), lambda i, j: (0, i, 0)),
        ],
        out_specs=pl.BlockSpec((BLK, H), lambda i, j: (i + (NP // BLK) * j, 0)),
        out_shape=jax.ShapeDtypeStruct((2 * NP, H), jnp.float32),
    )(h, dparts)


SB = 32


def _prop_body(gflat, srci, dsti, out, srcv, dstv, bufa, bufb, accs,
               sema, semb, semsa, semsb):
    c = lax.axis_index("c")
    s = lax.axis_index("s")
    off = c * NP

    @pl.loop(0, CW)
    def _(r):
        for j in range(8):
            bufa[r, pl.ds(j * 16, 16)] = jnp.zeros((16,), jnp.float32)

    @pl.loop(0, ROWS_PER_TILE // CW)
    def _(i):
        pltpu.sync_copy(
            bufa, accs.at[pl.ds(s * ROWS_PER_TILE + i * CW, CW)])

    plsc.subcore_barrier()

    def wait(buf, sem):
        pltpu.make_async_copy(gflat.at[pl.ds(0, CW)], buf, sem).wait()

    @pl.loop(0, CH // SB)
    def _(sb):
        st = pl.multiple_of(sb * SB, SB)
        pltpu.sync_copy(srci.at[s, pl.ds(st, SB)], srcv)
        pltpu.sync_copy(dsti.at[s, pl.ds(st, SB)], dstv)

        @pl.loop(0, SB)
        def _(ch):
            for j in range(8):
                sl = (ch, pl.ds(16 * j, 16))
                srcv[sl] = srcv[sl] + off

        pltpu.async_copy(gflat.at[srcv.at[0]], bufa, sema)
        pltpu.async_copy(gflat.at[srcv.at[1]], bufb, semb)

        @pl.loop(0, SB // 2)
        def _(i):
            j = i * 2
            wait(bufa, sema)
            pltpu.async_copy(bufa, accs.at[dstv.at[j]], semsa, add=True)
            wait(bufb, semb)
            pltpu.async_copy(bufb, accs.at[dstv.at[j + 1]], semsb, add=True)

            @pl.when(i < SB // 2 - 1)
            def _():
                pltpu.make_async_copy(
                    bufa, accs.at[pl.ds(0, CW)], semsa).wait()
                pltpu.async_copy(gflat.at[srcv.at[j + 2]], bufa, sema)
                pltpu.make_async_copy(
                    bufb, accs.at[pl.ds(0, CW)], semsb).wait()
                pltpu.async_copy(gflat.at[srcv.at[j + 3]], bufb, semb)

        pltpu.make_async_copy(bufa, accs.at[pl.ds(0, CW)], semsa).wait()
        pltpu.make_async_copy(bufb, accs.at[pl.ds(0, CW)], semsb).wait()

    plsc.subcore_barrier()
    pltpu.sync_copy(accs.at[pl.ds(s * ROWS_PER_TILE, ROWS_PER_TILE)],
                    out.at[pl.ds(c * NP + s * ROWS_PER_TILE, ROWS_PER_TILE)])


def _propagate(gflat, srci, dsti):
    mesh = plsc.VectorSubcoreMesh(core_axis_name="c", subcore_axis_name="s")
    k = pl.kernel(
        _prop_body,
        out_type=jax.ShapeDtypeStruct((NC * NP, H), jnp.float32),
        mesh=mesh,
        scratch_types=[
            pltpu.VMEM((SB, CW), jnp.int32),
            pltpu.VMEM((SB, CW), jnp.int32),
            pltpu.VMEM((CW, H), jnp.float32),
            pltpu.VMEM((CW, H), jnp.float32),
            pltpu.VMEM_SHARED((NP, H), jnp.float32),
            pltpu.SemaphoreType.DMA,
            pltpu.SemaphoreType.DMA,
            pltpu.SemaphoreType.DMA,
            pltpu.SemaphoreType.DMA,
        ],
    )
    return k(gflat, srci, dsti)


def _combine_body(acc_ref, g_ref, dp_ref, o_ref):
    cnt = dp_ref[0] + dp_ref[1] + 1.0
    dinv = lax.rsqrt(cnt)
    o_ref[...] = dinv * (acc_ref[...] + g_ref[...])


def _combine(accflat, gflat, dparts):
    return pl.pallas_call(
        _combine_body,
        grid=(NP // BLK, 2),
        in_specs=[
            pl.BlockSpec((BLK, H), lambda i, j: (i + (NP // BLK) * j, 0)),
            pl.BlockSpec((BLK, H), lambda i, j: (i + (NP // BLK) * j, 0)),
            pl.BlockSpec((2, BLK, 1), lambda i, j: (0, i, 0)),
        ],
        out_specs=pl.BlockSpec((BLK, H), lambda i, j: (i, j)),
        out_shape=jax.ShapeDtypeStruct((NP, Z_DIM), jnp.float32),
    )(accflat, gflat, dparts)


def kernel(x, edge_index, W, b):
    xp = jnp.pad(x, ((0, NP - N), (0, 0)))
    src = edge_index[0].astype(jnp.int32)
    dst = edge_index[1].astype(jnp.int32)
    pad_ids = jnp.arange(EPAD - E, dtype=jnp.int32)
    src_p = jnp.concatenate([src, pad_ids % N])
    dst_p = jnp.concatenate([dst, N + pad_ids % 128])
    srci = src_p.reshape(NS, CH, CW)
    dsti = dst_p.reshape(NS, CH, CW)

    h = _mm_norm(xp, W, b)
    dcounts = _deg_counts(dsti)
    dparts = dcounts.reshape(2, NP, 1)
    gflat = _scale(h, dparts)
    accflat = _propagate(gflat, srci, dsti)
    out = _combine(accflat, gflat, dparts)
    return out[:N]


# speedup vs baseline: 1.1117x; 1.1117x over previous
import functools

import jax
import jax.numpy as jnp
from jax import lax
from jax.experimental import pallas as pl
from jax.experimental.pallas import tpu as pltpu
from jax.experimental.pallas import tpu_sc as plsc

N = 10000
D_IN = 256
Z_DIM = 256
H = 128
NP = 10240
E = 320000
CW = 128
NS = 16
NC = 2
CH = 160
EPAD = NS * CH * CW
ROWS_PER_TILE = NP // NS
BLK = 1024


def _mm_norm_body(x_ref, w_ref, b_ref, h_ref):
    h = lax.dot_general(x_ref[...], w_ref[...],
                        dimension_numbers=(((1,), (1,)), ((), ())),
                        preferred_element_type=jnp.float32)
    h = h + b_ref[...]
    l2 = jnp.sqrt(jnp.sum(h * h, axis=1, keepdims=True))
    h_ref[...] = h / jnp.maximum(l2, 1e-12) * 1.8


def _mm_norm(xp, W, b):
    return pl.pallas_call(
        _mm_norm_body,
        grid=(NP // BLK,),
        in_specs=[
            pl.BlockSpec((BLK, D_IN), lambda i: (i, 0)),
            pl.BlockSpec((Z_DIM, D_IN), lambda i: (0, 0)),
            pl.BlockSpec((1, Z_DIM), lambda i: (0, 0)),
        ],
        out_specs=pl.BlockSpec((BLK, Z_DIM), lambda i: (i, 0)),
        out_shape=jax.ShapeDtypeStruct((NP, Z_DIM), jnp.float32),
    )(xp, W, b.reshape(1, Z_DIM))


def _deg_body(dsti, out, dstv, onesb, zb, dacc):
    c = lax.axis_index("c")
    s = lax.axis_index("s")
    pltpu.sync_copy(dsti.at[s, pl.ds((CH // 2) * c, CH // 2)], dstv)
    for j in range(8):
        onesb[pl.ds(16 * j, 16)] = jnp.ones((16,), jnp.float32)

    @pl.loop(0, ROWS_PER_TILE, step=16)
    def _(r):
        zb[pl.ds(r, 16)] = jnp.zeros((16,), jnp.float32)

    pltpu.sync_copy(zb, dacc.at[pl.ds(s * ROWS_PER_TILE, ROWS_PER_TILE)])
    plsc.subcore_barrier()

    @pl.loop(0, CH // 2)
    def _(ch):
        pltpu.sync_copy(onesb, dacc.at[dstv.at[ch]], add=True)

    plsc.subcore_barrier()
    pltpu.sync_copy(dacc.at[pl.ds(s * ROWS_PER_TILE, ROWS_PER_TILE)],
                    out.at[pl.ds(c * NP + s * ROWS_PER_TILE, ROWS_PER_TILE)])


def _deg_counts(dsti):
    mesh = plsc.VectorSubcoreMesh(core_axis_name="c", subcore_axis_name="s")
    k = pl.kernel(
        _deg_body,
        out_type=jax.ShapeDtypeStruct((NC * NP,), jnp.float32),
        mesh=mesh,
        scratch_types=[
            pltpu.VMEM((CH // 2, CW), jnp.int32),
            pltpu.VMEM((CW,), jnp.float32),
            pltpu.VMEM((ROWS_PER_TILE,), jnp.float32),
            pltpu.VMEM_SHARED((NP,), jnp.float32),
        ],
    )
    return k(dsti)


def _scale_body(h_ref, dp_ref, g_ref):
    cnt = dp_ref[0] + dp_ref[1] + 1.0
    dinv = lax.rsqrt(cnt)
    g_ref[...] = h_ref[...] * dinv


def _scale(h, dparts):
    return pl.pallas_call(
        _scale_body,
        grid=(NP // BLK, 2),
        in_specs=[
            pl.BlockSpec((BLK, H), lambda i, j: (i, j)),
            pl.BlockSpec((2, BLK, 1), lambda i, j: (0, i, 0)),
        ],
        out_specs=pl.BlockSpec((BLK, H), lambda i, j: (i + (NP // BLK) * j, 0)),
        out_shape=jax.ShapeDtypeStruct((2 * NP, H), jnp.float32),
    )(h, dparts)


SB = 32


def _prop_body(gflat, srci, dsti, out, srcv, dstv, bufa, bufb, accs,
               sema, semb):
    c = lax.axis_index("c")
    s = lax.axis_index("s")
    off = c * NP

    @pl.loop(0, CW)
    def _(r):
        for j in range(8):
            bufa[r, pl.ds(j * 16, 16)] = jnp.zeros((16,), jnp.float32)

    @pl.loop(0, ROWS_PER_TILE // CW)
    def _(i):
        pltpu.sync_copy(
            bufa, accs.at[pl.ds(s * ROWS_PER_TILE + i * CW, CW)])

    plsc.subcore_barrier()

    def wait(buf, sem):
        pltpu.make_async_copy(gflat.at[pl.ds(0, CW)], buf, sem).wait()

    @pl.loop(0, CH // SB)
    def _(sb):
        st = pl.multiple_of(sb * SB, SB)
        pltpu.sync_copy(srci.at[s, pl.ds(st, SB)], srcv)
        pltpu.sync_copy(dsti.at[s, pl.ds(st, SB)], dstv)

        @pl.loop(0, SB)
        def _(ch):
            for j in range(8):
                sl = (ch, pl.ds(16 * j, 16))
                srcv[sl] = srcv[sl] + off

        pltpu.async_copy(gflat.at[srcv.at[0]], bufa, sema)

        @pl.loop(0, SB // 2)
        def _(i):
            j = i * 2
            wait(bufa, sema)
            pltpu.async_copy(gflat.at[srcv.at[j + 1]], bufb, semb)
            pltpu.sync_copy(bufa, accs.at[dstv.at[j]], add=True)
            wait(bufb, semb)

            @pl.when(i < SB // 2 - 1)
            def _():
                pltpu.async_copy(gflat.at[srcv.at[j + 2]], bufa, sema)

            pltpu.sync_copy(bufb, accs.at[dstv.at[j + 1]], add=True)

    plsc.subcore_barrier()
    pltpu.sync_copy(accs.at[pl.ds(s * ROWS_PER_TILE, ROWS_PER_TILE)],
                    out.at[pl.ds(c * NP + s * ROWS_PER_TILE, ROWS_PER_TILE)])


def _propagate(gflat, srci, dsti):
    mesh = plsc.VectorSubcoreMesh(core_axis_name="c", subcore_axis_name="s")
    k = pl.kernel(
        _prop_body,
        out_type=jax.ShapeDtypeStruct((NC * NP, H), jnp.float32),
        mesh=mesh,
        scratch_types=[
            pltpu.VMEM((SB, CW), jnp.int32),
            pltpu.VMEM((SB, CW), jnp.int32),
            pltpu.VMEM((CW, H), jnp.float32),
            pltpu.VMEM((CW, H), jnp.float32),
            pltpu.VMEM_SHARED((NP, H), jnp.float32),
            pltpu.SemaphoreType.DMA,
            pltpu.SemaphoreType.DMA,
        ],
    )
    return k(gflat, srci, dsti)


def _combine_body(acc_ref, g_ref, dp_ref, o_ref):
    cnt = dp_ref[0] + dp_ref[1] + 1.0
    dinv = lax.rsqrt(cnt)
    o_ref[...] = dinv * (acc_ref[...] + g_ref[...])


def _combine(accflat, gflat, dparts):
    return pl.pallas_call(
        _combine_body,
        grid=(NP // BLK, 2),
        in_specs=[
            pl.BlockSpec((BLK, H), lambda i, j: (i + (NP // BLK) * j, 0)),
            pl.BlockSpec((BLK, H), lambda i, j: (i + (NP // BLK) * j, 0)),
            pl.BlockSpec((2, BLK, 1), lambda i, j: (0, i, 0)),
        ],
        out_specs=pl.BlockSpec((BLK, H), lambda i, j: (i, j)),
        out_shape=jax.ShapeDtypeStruct((N, Z_DIM), jnp.float32),
    )(accflat, gflat, dparts)


def kernel(x, edge_index, W, b):
    xp = jnp.pad(x, ((0, NP - N), (0, 0)))
    src = edge_index[0].astype(jnp.int32)
    dst = edge_index[1].astype(jnp.int32)
    pad_ids = jnp.arange(EPAD - E, dtype=jnp.int32)
    src_p = jnp.concatenate([src, pad_ids % N])
    dst_p = jnp.concatenate([dst, N + pad_ids % 128])
    srci = src_p.reshape(NS, CH, CW)
    dsti = dst_p.reshape(NS, CH, CW)

    h = _mm_norm(xp, W, b)
    dcounts = _deg_counts(dsti)
    dparts = dcounts.reshape(2, NP, 1)
    gflat = _scale(h, dparts)
    accflat = _propagate(gflat, srci, dsti)
    return _combine(accflat, gflat, dparts)


# speedup vs baseline: 1.2893x; 1.1597x over previous
import functools

import jax
import jax.numpy as jnp
from jax import lax
from jax.experimental import pallas as pl
from jax.experimental.pallas import tpu as pltpu
from jax.experimental.pallas import tpu_sc as plsc

N = 10000
D_IN = 256
Z_DIM = 256
H = 128
NP = 10240
E = 320000
CW = 128
NS = 16
NC = 2
CH = 160
EPAD = NS * CH * CW
CWD = 96
CHD = 216
EPADD = NS * CHD * CWD
ROWS_PER_TILE = NP // NS
BLK = 1024


def _mm_norm_body(x_ref, w_ref, b_ref, h_ref):
    h = lax.dot_general(x_ref[...], w_ref[...],
                        dimension_numbers=(((1,), (1,)), ((), ())),
                        preferred_element_type=jnp.float32)
    h = h + b_ref[...]
    l2 = jnp.sqrt(jnp.sum(h * h, axis=1, keepdims=True))
    h_ref[...] = h / jnp.maximum(l2, 1e-12) * 1.8


def _mm_norm(xp, W, b):
    return pl.pallas_call(
        _mm_norm_body,
        grid=(NP // BLK,),
        in_specs=[
            pl.BlockSpec((BLK, D_IN), lambda i: (i, 0)),
            pl.BlockSpec((Z_DIM, D_IN), lambda i: (0, 0)),
            pl.BlockSpec((1, Z_DIM), lambda i: (0, 0)),
        ],
        out_specs=pl.BlockSpec((BLK, Z_DIM), lambda i: (i, 0)),
        out_shape=jax.ShapeDtypeStruct((NP, Z_DIM), jnp.float32),
    )(xp, W, b.reshape(1, Z_DIM))


def _deg_body(dsti, out, dstv, onesb, zb, dacc):
    c = lax.axis_index("c")
    s = lax.axis_index("s")
    pltpu.sync_copy(dsti.at[s, pl.ds((CH // 2) * c, CH // 2)], dstv)
    for j in range(8):
        onesb[pl.ds(16 * j, 16)] = jnp.ones((16,), jnp.float32)

    @pl.loop(0, ROWS_PER_TILE, step=16)
    def _(r):
        zb[pl.ds(r, 16)] = jnp.zeros((16,), jnp.float32)

    pltpu.sync_copy(zb, dacc.at[pl.ds(s * ROWS_PER_TILE, ROWS_PER_TILE)])
    plsc.subcore_barrier()

    @pl.loop(0, CH // 2)
    def _(ch):
        pltpu.sync_copy(onesb, dacc.at[dstv.at[ch]], add=True)

    plsc.subcore_barrier()
    pltpu.sync_copy(dacc.at[pl.ds(s * ROWS_PER_TILE, ROWS_PER_TILE)],
                    out.at[pl.ds(c * NP + s * ROWS_PER_TILE, ROWS_PER_TILE)])


def _deg_counts(dsti):
    mesh = plsc.VectorSubcoreMesh(core_axis_name="c", subcore_axis_name="s")
    k = pl.kernel(
        _deg_body,
        out_type=jax.ShapeDtypeStruct((NC * NP,), jnp.float32),
        mesh=mesh,
        scratch_types=[
            pltpu.VMEM((CH // 2, CW), jnp.int32),
            pltpu.VMEM((CW,), jnp.float32),
            pltpu.VMEM((ROWS_PER_TILE,), jnp.float32),
            pltpu.VMEM_SHARED((NP,), jnp.float32),
        ],
    )
    return k(dsti)


def _scale_body(h_ref, dp_ref, g_ref):
    cnt = dp_ref[0] + dp_ref[1] + 1.0
    dinv = lax.rsqrt(cnt)
    g_ref[...] = h_ref[...] * dinv


def _scale(h, dparts):
    return pl.pallas_call(
        _scale_body,
        grid=(NP // BLK, 2),
        in_specs=[
            pl.BlockSpec((BLK, H), lambda i, j: (i, j)),
            pl.BlockSpec((2, BLK, 1), lambda i, j: (0, i, 0)),
        ],
        out_specs=pl.BlockSpec((BLK, H), lambda i, j: (i + (NP // BLK) * j, 0)),
        out_shape=jax.ShapeDtypeStruct((2 * NP, H), jnp.float32),
    )(h, dparts)


SB = 24


def _prop_body(gflat, srci, dsti, out, srcv, dstv, bufa, bufb, bufc, accs,
               sema, semb, semc):
    c = lax.axis_index("c")
    s = lax.axis_index("s")
    off = c * NP

    @pl.loop(0, CWD)
    def _(r):
        for j in range(8):
            bufa[r, pl.ds(j * 16, 16)] = jnp.zeros((16,), jnp.float32)

    @pl.loop(0, ROWS_PER_TILE // CWD + 1)
    def _(i):
        st = i * CWD

        @pl.when(st + CWD <= ROWS_PER_TILE)
        def _():
            pltpu.sync_copy(
                bufa, accs.at[pl.ds(s * ROWS_PER_TILE + st, CWD)])

        @pl.when(st + CWD > ROWS_PER_TILE)
        def _():
            pltpu.sync_copy(
                bufa.at[pl.ds(0, ROWS_PER_TILE - (ROWS_PER_TILE // CWD) * CWD)],
                accs.at[pl.ds(
                    s * ROWS_PER_TILE + (ROWS_PER_TILE // CWD) * CWD,
                    ROWS_PER_TILE - (ROWS_PER_TILE // CWD) * CWD)])

    plsc.subcore_barrier()

    def wait(buf, sem):
        pltpu.make_async_copy(gflat.at[pl.ds(0, CWD)], buf, sem).wait()

    @pl.loop(0, CHD // SB)
    def _(sb):
        st = pl.multiple_of(sb * SB, SB)
        pltpu.sync_copy(srci.at[s, pl.ds(st, SB)], srcv)
        pltpu.sync_copy(dsti.at[s, pl.ds(st, SB)], dstv)

        @pl.loop(0, SB)
        def _(ch):
            for j in range(6):
                sl = (ch, pl.ds(16 * j, 16))
                srcv[sl] = srcv[sl] + off

        pltpu.async_copy(gflat.at[srcv.at[0]], bufa, sema)
        pltpu.async_copy(gflat.at[srcv.at[1]], bufb, semb)

        @pl.loop(0, SB // 3)
        def _(i):
            j = i * 3
            wait(bufa, sema)
            pltpu.async_copy(gflat.at[srcv.at[j + 2]], bufc, semc)
            pltpu.sync_copy(bufa, accs.at[dstv.at[j]], add=True)
            wait(bufb, semb)

            @pl.when(j + 3 < SB)
            def _():
                pltpu.async_copy(gflat.at[srcv.at[j + 3]], bufa, sema)

            pltpu.sync_copy(bufb, accs.at[dstv.at[j + 1]], add=True)
            wait(bufc, semc)

            @pl.when(j + 4 < SB)
            def _():
                pltpu.async_copy(gflat.at[srcv.at[j + 4]], bufb, semb)

            pltpu.sync_copy(bufc, accs.at[dstv.at[j + 2]], add=True)

    plsc.subcore_barrier()
    pltpu.sync_copy(accs.at[pl.ds(s * ROWS_PER_TILE, ROWS_PER_TILE)],
                    out.at[pl.ds(c * NP + s * ROWS_PER_TILE, ROWS_PER_TILE)])


def _propagate(gflat, srci, dsti):
    mesh = plsc.VectorSubcoreMesh(core_axis_name="c", subcore_axis_name="s")
    k = pl.kernel(
        _prop_body,
        out_type=jax.ShapeDtypeStruct((NC * NP, H), jnp.float32),
        mesh=mesh,
        scratch_types=[
            pltpu.VMEM((SB, CWD), jnp.int32),
            pltpu.VMEM((SB, CWD), jnp.int32),
            pltpu.VMEM((CWD, H), jnp.float32),
            pltpu.VMEM((CWD, H), jnp.float32),
            pltpu.VMEM((CWD, H), jnp.float32),
            pltpu.VMEM_SHARED((NP, H), jnp.float32),
            pltpu.SemaphoreType.DMA,
            pltpu.SemaphoreType.DMA,
            pltpu.SemaphoreType.DMA,
        ],
    )
    return k(gflat, srci, dsti)


def _combine_body(acc_ref, g_ref, dp_ref, o_ref):
    cnt = dp_ref[0] + dp_ref[1] + 1.0
    dinv = lax.rsqrt(cnt)
    o_ref[...] = dinv * (acc_ref[...] + g_ref[...])


def _combine(accflat, gflat, dparts):
    return pl.pallas_call(
        _combine_body,
        grid=(NP // BLK, 2),
        in_specs=[
            pl.BlockSpec((BLK, H), lambda i, j: (i + (NP // BLK) * j, 0)),
            pl.BlockSpec((BLK, H), lambda i, j: (i + (NP // BLK) * j, 0)),
            pl.BlockSpec((2, BLK, 1), lambda i, j: (0, i, 0)),
        ],
        out_specs=pl.BlockSpec((BLK, H), lambda i, j: (i, j)),
        out_shape=jax.ShapeDtypeStruct((N, Z_DIM), jnp.float32),
    )(accflat, gflat, dparts)


def kernel(x, edge_index, W, b):
    xp = jnp.pad(x, ((0, NP - N), (0, 0)))
    src = edge_index[0].astype(jnp.int32)
    dst = edge_index[1].astype(jnp.int32)
    pad_b = jnp.arange(EPAD - E, dtype=jnp.int32)
    dsti = jnp.concatenate([dst, N + pad_b % 128]).reshape(NS, CH, CW)
    pad_d = jnp.arange(EPADD - E, dtype=jnp.int32)
    srci_d = jnp.concatenate([src, pad_d % N]).reshape(NS, CHD, CWD)
    dsti_d = jnp.concatenate([dst, N + pad_d % 128]).reshape(NS, CHD, CWD)

    h = _mm_norm(xp, W, b)
    dcounts = _deg_counts(dsti)
    dparts = dcounts.reshape(2, NP, 1)
    gflat = _scale(h, dparts)
    accflat = _propagate(gflat, srci_d, dsti_d)
    return _combine(accflat, gflat, dparts)


# speedup vs baseline: 1.3072x; 1.0139x over previous
import functools

import jax
import jax.numpy as jnp
from jax import lax
from jax.experimental import pallas as pl
from jax.experimental.pallas import tpu as pltpu
from jax.experimental.pallas import tpu_sc as plsc

N = 10000
D_IN = 256
Z_DIM = 256
H = 128
NP = 10240
E = 320000
CW = 128
NS = 16
NC = 2
CH = 160
EPAD = NS * CH * CW
CWD = 80
CHD = 256
EPADD = NS * CHD * CWD
ROWS_PER_TILE = NP // NS
BLK = 1024


def _mm_norm_body(x_ref, w_ref, b_ref, h_ref):
    h = lax.dot_general(x_ref[...], w_ref[...],
                        dimension_numbers=(((1,), (1,)), ((), ())),
                        preferred_element_type=jnp.float32)
    h = h + b_ref[...]
    l2 = jnp.sqrt(jnp.sum(h * h, axis=1, keepdims=True))
    h_ref[...] = h / jnp.maximum(l2, 1e-12) * 1.8


def _mm_norm(xp, W, b):
    return pl.pallas_call(
        _mm_norm_body,
        grid=(NP // BLK,),
        in_specs=[
            pl.BlockSpec((BLK, D_IN), lambda i: (i, 0)),
            pl.BlockSpec((Z_DIM, D_IN), lambda i: (0, 0)),
            pl.BlockSpec((1, Z_DIM), lambda i: (0, 0)),
        ],
        out_specs=pl.BlockSpec((BLK, Z_DIM), lambda i: (i, 0)),
        out_shape=jax.ShapeDtypeStruct((NP, Z_DIM), jnp.float32),
    )(xp, W, b.reshape(1, Z_DIM))


def _deg_body(dsti, out, dstv, onesb, zb, dacc):
    c = lax.axis_index("c")
    s = lax.axis_index("s")
    pltpu.sync_copy(dsti.at[s, pl.ds((CH // 2) * c, CH // 2)], dstv)
    for j in range(8):
        onesb[pl.ds(16 * j, 16)] = jnp.ones((16,), jnp.float32)

    @pl.loop(0, ROWS_PER_TILE, step=16)
    def _(r):
        zb[pl.ds(r, 16)] = jnp.zeros((16,), jnp.float32)

    pltpu.sync_copy(zb, dacc.at[pl.ds(s * ROWS_PER_TILE, ROWS_PER_TILE)])
    plsc.subcore_barrier()

    @pl.loop(0, CH // 2)
    def _(ch):
        pltpu.sync_copy(onesb, dacc.at[dstv.at[ch]], add=True)

    plsc.subcore_barrier()
    pltpu.sync_copy(dacc.at[pl.ds(s * ROWS_PER_TILE, ROWS_PER_TILE)],
                    out.at[pl.ds(c * NP + s * ROWS_PER_TILE, ROWS_PER_TILE)])


def _deg_counts(dsti):
    mesh = plsc.VectorSubcoreMesh(core_axis_name="c", subcore_axis_name="s")
    k = pl.kernel(
        _deg_body,
        out_type=jax.ShapeDtypeStruct((NC * NP,), jnp.float32),
        mesh=mesh,
        scratch_types=[
            pltpu.VMEM((CH // 2, CW), jnp.int32),
            pltpu.VMEM((CW,), jnp.float32),
            pltpu.VMEM((ROWS_PER_TILE,), jnp.float32),
            pltpu.VMEM_SHARED((NP,), jnp.float32),
        ],
    )
    return k(dsti)


def _scale_body(h_ref, dp_ref, g_ref):
    cnt = dp_ref[0] + dp_ref[1] + 1.0
    dinv = lax.rsqrt(cnt)
    g_ref[...] = h_ref[...] * dinv


def _scale(h, dparts):
    return pl.pallas_call(
        _scale_body,
        grid=(NP // BLK, 2),
        in_specs=[
            pl.BlockSpec((BLK, H), lambda i, j: (i, j)),
            pl.BlockSpec((2, BLK, 1), lambda i, j: (0, i, 0)),
        ],
        out_specs=pl.BlockSpec((BLK, H), lambda i, j: (i + (NP // BLK) * j, 0)),
        out_shape=jax.ShapeDtypeStruct((2 * NP, H), jnp.float32),
    )(h, dparts)


SB = 32


def _prop_body(gflat, srci, dsti, out, srcv, dstv, bufa, bufb, bufc, bufd,
               accs, sema, semb, semc, semd):
    c = lax.axis_index("c")
    s = lax.axis_index("s")
    off = c * NP

    @pl.loop(0, CWD)
    def _(r):
        for j in range(8):
            bufa[r, pl.ds(j * 16, 16)] = jnp.zeros((16,), jnp.float32)

    @pl.loop(0, ROWS_PER_TILE // CWD)
    def _(i):
        pltpu.sync_copy(
            bufa, accs.at[pl.ds(s * ROWS_PER_TILE + i * CWD, CWD)])

    plsc.subcore_barrier()

    def wait(buf, sem):
        pltpu.make_async_copy(gflat.at[pl.ds(0, CWD)], buf, sem).wait()

    @pl.loop(0, CHD // SB)
    def _(sb):
        st = pl.multiple_of(sb * SB, SB)
        pltpu.sync_copy(srci.at[s, pl.ds(st, SB)], srcv)
        pltpu.sync_copy(dsti.at[s, pl.ds(st, SB)], dstv)

        @pl.loop(0, SB)
        def _(ch):
            for j in range(5):
                sl = (ch, pl.ds(16 * j, 16))
                srcv[sl] = srcv[sl] + off

        pltpu.async_copy(gflat.at[srcv.at[0]], bufa, sema)
        pltpu.async_copy(gflat.at[srcv.at[1]], bufb, semb)
        pltpu.async_copy(gflat.at[srcv.at[2]], bufc, semc)

        @pl.loop(0, SB // 4)
        def _(i):
            j = i * 4
            wait(bufa, sema)
            pltpu.async_copy(gflat.at[srcv.at[j + 3]], bufd, semd)
            pltpu.sync_copy(bufa, accs.at[dstv.at[j]], add=True)
            wait(bufb, semb)

            @pl.when(j + 4 < SB)
            def _():
                pltpu.async_copy(gflat.at[srcv.at[j + 4]], bufa, sema)

            pltpu.sync_copy(bufb, accs.at[dstv.at[j + 1]], add=True)
            wait(bufc, semc)

            @pl.when(j + 5 < SB)
            def _():
                pltpu.async_copy(gflat.at[srcv.at[j + 5]], bufb, semb)

            pltpu.sync_copy(bufc, accs.at[dstv.at[j + 2]], add=True)
            wait(bufd, semd)

            @pl.when(j + 6 < SB)
            def _():
                pltpu.async_copy(gflat.at[srcv.at[j + 6]], bufc, semc)

            pltpu.sync_copy(bufd, accs.at[dstv.at[j + 3]], add=True)

    plsc.subcore_barrier()
    pltpu.sync_copy(accs.at[pl.ds(s * ROWS_PER_TILE, ROWS_PER_TILE)],
                    out.at[pl.ds(c * NP + s * ROWS_PER_TILE, ROWS_PER_TILE)])


def _propagate(gflat, srci, dsti):
    mesh = plsc.VectorSubcoreMesh(core_axis_name="c", subcore_axis_name="s")
    k = pl.kernel(
        _prop_body,
        out_type=jax.ShapeDtypeStruct((NC * NP, H), jnp.float32),
        mesh=mesh,
        scratch_types=[
            pltpu.VMEM((SB, CWD), jnp.int32),
            pltpu.VMEM((SB, CWD), jnp.int32),
            pltpu.VMEM((CWD, H), jnp.float32),
            pltpu.VMEM((CWD, H), jnp.float32),
            pltpu.VMEM((CWD, H), jnp.float32),
            pltpu.VMEM((CWD, H), jnp.float32),
            pltpu.VMEM_SHARED((NP, H), jnp.float32),
            pltpu.SemaphoreType.DMA,
            pltpu.SemaphoreType.DMA,
            pltpu.SemaphoreType.DMA,
            pltpu.SemaphoreType.DMA,
        ],
    )
    return k(gflat, srci, dsti)


def _combine_body(acc_ref, g_ref, dp_ref, o_ref):
    cnt = dp_ref[0] + dp_ref[1] + 1.0
    dinv = lax.rsqrt(cnt)
    o_ref[...] = dinv * (acc_ref[...] + g_ref[...])


def _combine(accflat, gflat, dparts):
    return pl.pallas_call(
        _combine_body,
        grid=(NP // BLK, 2),
        in_specs=[
            pl.BlockSpec((BLK, H), lambda i, j: (i + (NP // BLK) * j, 0)),
            pl.BlockSpec((BLK, H), lambda i, j: (i + (NP // BLK) * j, 0)),
            pl.BlockSpec((2, BLK, 1), lambda i, j: (0, i, 0)),
        ],
        out_specs=pl.BlockSpec((BLK, H), lambda i, j: (i, j)),
        out_shape=jax.ShapeDtypeStruct((N, Z_DIM), jnp.float32),
    )(accflat, gflat, dparts)


def kernel(x, edge_index, W, b):
    xp = jnp.pad(x, ((0, NP - N), (0, 0)))
    src = edge_index[0].astype(jnp.int32)
    dst = edge_index[1].astype(jnp.int32)
    pad_b = jnp.arange(EPAD - E, dtype=jnp.int32)
    dsti = jnp.concatenate([dst, N + pad_b % 128]).reshape(NS, CH, CW)
    pad_d = jnp.arange(EPADD - E, dtype=jnp.int32)
    srci_d = jnp.concatenate([src, pad_d % N]).reshape(NS, CHD, CWD)
    dsti_d = jnp.concatenate([dst, N + pad_d % 128]).reshape(NS, CHD, CWD)

    h = _mm_norm(xp, W, b)
    dcounts = _deg_counts(dsti)
    dparts = dcounts.reshape(2, NP, 1)
    gflat = _scale(h, dparts)
    accflat = _propagate(gflat, srci_d, dsti_d)
    return _combine(accflat, gflat, dparts)


# speedup vs baseline: 1.3449x; 1.0288x over previous
import functools

import jax
import jax.numpy as jnp
from jax import lax
from jax.experimental import pallas as pl
from jax.experimental.pallas import tpu as pltpu
from jax.experimental.pallas import tpu_sc as plsc

N = 10000
D_IN = 256
Z_DIM = 256
H = 128
NP = 10240
E = 320000
CW = 128
NS = 16
NC = 2
CH = 160
EPAD = NS * CH * CW
CWD = 80
CHD = 256
EPADD = NS * CHD * CWD
ROWS_PER_TILE = NP // NS
BLK = 1024


def _mmscale_body(x_ref, w_ref, b_ref, dp_ref, g_ref, hi_ref):
    j = pl.program_id(1)

    @pl.when(j == 0)
    def _():
        h = lax.dot_general(x_ref[...], w_ref[...],
                            dimension_numbers=(((1,), (1,)), ((), ())),
                            preferred_element_type=jnp.float32)
        h = h + b_ref[...]
        l2 = jnp.sqrt(jnp.sum(h * h, axis=1, keepdims=True))
        h = h / jnp.maximum(l2, 1e-12) * 1.8
        cnt = dp_ref[0] + dp_ref[1] + 1.0
        g = h * lax.rsqrt(cnt)
        g_ref[...] = g[:, :H]
        hi_ref[...] = g[:, H:]

    @pl.when(j == 1)
    def _():
        g_ref[...] = hi_ref[...]


def _mmscale(x, W, b, dparts):
    return pl.pallas_call(
        _mmscale_body,
        grid=(NP // BLK, 2),
        in_specs=[
            pl.BlockSpec((BLK, D_IN), lambda i, j: (i, 0)),
            pl.BlockSpec((Z_DIM, D_IN), lambda i, j: (0, 0)),
            pl.BlockSpec((1, Z_DIM), lambda i, j: (0, 0)),
            pl.BlockSpec((2, BLK, 1), lambda i, j: (0, i, 0)),
        ],
        out_specs=pl.BlockSpec((BLK, H), lambda i, j: (i + (NP // BLK) * j, 0)),
        out_shape=jax.ShapeDtypeStruct((2 * NP, H), jnp.float32),
        scratch_shapes=[pltpu.VMEM((BLK, H), jnp.float32)],
    )(x, W, b.reshape(1, Z_DIM), dparts)


def _deg_body(dsti, out, dstv, onesb, zb, dacc):
    c = lax.axis_index("c")
    s = lax.axis_index("s")
    pltpu.sync_copy(dsti.at[s, pl.ds((CH // 2) * c, CH // 2)], dstv)
    for j in range(8):
        onesb[pl.ds(16 * j, 16)] = jnp.ones((16,), jnp.float32)

    @pl.loop(0, ROWS_PER_TILE, step=16)
    def _(r):
        zb[pl.ds(r, 16)] = jnp.zeros((16,), jnp.float32)

    pltpu.sync_copy(zb, dacc.at[pl.ds(s * ROWS_PER_TILE, ROWS_PER_TILE)])
    plsc.subcore_barrier()

    @pl.loop(0, CH // 2)
    def _(ch):
        pltpu.sync_copy(onesb, dacc.at[dstv.at[ch]], add=True)

    plsc.subcore_barrier()
    pltpu.sync_copy(dacc.at[pl.ds(s * ROWS_PER_TILE, ROWS_PER_TILE)],
                    out.at[pl.ds(c * NP + s * ROWS_PER_TILE, ROWS_PER_TILE)])


def _deg_counts(dsti):
    mesh = plsc.VectorSubcoreMesh(core_axis_name="c", subcore_axis_name="s")
    k = pl.kernel(
        _deg_body,
        out_type=jax.ShapeDtypeStruct((NC * NP,), jnp.float32),
        mesh=mesh,
        scratch_types=[
            pltpu.VMEM((CH // 2, CW), jnp.int32),
            pltpu.VMEM((CW,), jnp.float32),
            pltpu.VMEM((ROWS_PER_TILE,), jnp.float32),
            pltpu.VMEM_SHARED((NP,), jnp.float32),
        ],
    )
    return k(dsti)


SB = 32


def _prop_body(gflat, srci, dsti, out, srcv, dstv, bufa, bufb, bufc, bufd,
               accs, sema, semb, semc, semd):
    c = lax.axis_index("c")
    s = lax.axis_index("s")
    off = c * NP

    @pl.loop(0, CWD)
    def _(r):
        for j in range(8):
            bufa[r, pl.ds(j * 16, 16)] = jnp.zeros((16,), jnp.float32)

    @pl.loop(0, ROWS_PER_TILE // CWD)
    def _(i):
        pltpu.sync_copy(
            bufa, accs.at[pl.ds(s * ROWS_PER_TILE + i * CWD, CWD)])

    plsc.subcore_barrier()

    def wait(buf, sem):
        pltpu.make_async_copy(gflat.at[pl.ds(0, CWD)], buf, sem).wait()

    @pl.loop(0, CHD // SB)
    def _(sb):
        st = pl.multiple_of(sb * SB, SB)
        pltpu.sync_copy(srci.at[s, pl.ds(st, SB)], srcv)
        pltpu.sync_copy(dsti.at[s, pl.ds(st, SB)], dstv)

        @pl.loop(0, SB)
        def _(ch):
            for j in range(5):
                sl = (ch, pl.ds(16 * j, 16))
                srcv[sl] = srcv[sl] + off

        pltpu.async_copy(gflat.at[srcv.at[0]], bufa, sema)
        pltpu.async_copy(gflat.at[srcv.at[1]], bufb, semb)
        pltpu.async_copy(gflat.at[srcv.at[2]], bufc, semc)

        @pl.loop(0, SB // 4)
        def _(i):
            j = i * 4
            wait(bufa, sema)
            pltpu.async_copy(gflat.at[srcv.at[j + 3]], bufd, semd)
            pltpu.sync_copy(bufa, accs.at[dstv.at[j]], add=True)
            wait(bufb, semb)

            @pl.when(j + 4 < SB)
            def _():
                pltpu.async_copy(gflat.at[srcv.at[j + 4]], bufa, sema)

            pltpu.sync_copy(bufb, accs.at[dstv.at[j + 1]], add=True)
            wait(bufc, semc)

            @pl.when(j + 5 < SB)
            def _():
                pltpu.async_copy(gflat.at[srcv.at[j + 5]], bufb, semb)

            pltpu.sync_copy(bufc, accs.at[dstv.at[j + 2]], add=True)
            wait(bufd, semd)

            @pl.when(j + 6 < SB)
            def _():
                pltpu.async_copy(gflat.at[srcv.at[j + 6]], bufc, semc)

            pltpu.sync_copy(bufd, accs.at[dstv.at[j + 3]], add=True)

    plsc.subcore_barrier()
    pltpu.sync_copy(accs.at[pl.ds(s * ROWS_PER_TILE, ROWS_PER_TILE)],
                    out.at[pl.ds(c * NP + s * ROWS_PER_TILE, ROWS_PER_TILE)])


def _propagate(gflat, srci, dsti):
    mesh = plsc.VectorSubcoreMesh(core_axis_name="c", subcore_axis_name="s")
    k = pl.kernel(
        _prop_body,
        out_type=jax.ShapeDtypeStruct((NC * NP, H), jnp.float32),
        mesh=mesh,
        scratch_types=[
            pltpu.VMEM((SB, CWD), jnp.int32),
            pltpu.VMEM((SB, CWD), jnp.int32),
            pltpu.VMEM((CWD, H), jnp.float32),
            pltpu.VMEM((CWD, H), jnp.float32),
            pltpu.VMEM((CWD, H), jnp.float32),
            pltpu.VMEM((CWD, H), jnp.float32),
            pltpu.VMEM_SHARED((NP, H), jnp.float32),
            pltpu.SemaphoreType.DMA,
            pltpu.SemaphoreType.DMA,
            pltpu.SemaphoreType.DMA,
            pltpu.SemaphoreType.DMA,
        ],
    )
    return k(gflat, srci, dsti)


def _combine_body(acc_ref, g_ref, dp_ref, o_ref):
    cnt = dp_ref[0] + dp_ref[1] + 1.0
    dinv = lax.rsqrt(cnt)
    o_ref[...] = dinv * (acc_ref[...] + g_ref[...])


def _combine(accflat, gflat, dparts):
    return pl.pallas_call(
        _combine_body,
        grid=(NP // BLK, 2),
        in_specs=[
            pl.BlockSpec((BLK, H), lambda i, j: (i + (NP // BLK) * j, 0)),
            pl.BlockSpec((BLK, H), lambda i, j: (i + (NP // BLK) * j, 0)),
            pl.BlockSpec((2, BLK, 1), lambda i, j: (0, i, 0)),
        ],
        out_specs=pl.BlockSpec((BLK, H), lambda i, j: (i, j)),
        out_shape=jax.ShapeDtypeStruct((N, Z_DIM), jnp.float32),
    )(accflat, gflat, dparts)


def kernel(x, edge_index, W, b):
    src = edge_index[0].astype(jnp.int32)
    dst = edge_index[1].astype(jnp.int32)
    pad_b = jnp.arange(EPAD - E, dtype=jnp.int32)
    dsti = jnp.concatenate([dst, N + pad_b % 128]).reshape(NS, CH, CW)
    pad_d = jnp.arange(EPADD - E, dtype=jnp.int32)
    srci_d = jnp.concatenate([src, pad_d % N]).reshape(NS, CHD, CWD)
    dsti_d = jnp.concatenate([dst, N + pad_d % 128]).reshape(NS, CHD, CWD)

    dcounts = _deg_counts(dsti)
    dparts = dcounts.reshape(2, NP, 1)
    gflat = _mmscale(x, W, b, dparts)
    accflat = _propagate(gflat, srci_d, dsti_d)
    return _combine(accflat, gflat, dparts)


# speedup vs baseline: 1.3480x; 1.0023x over previous
import functools

import jax
import jax.numpy as jnp
from jax import lax
from jax.experimental import pallas as pl
from jax.experimental.pallas import tpu as pltpu
from jax.experimental.pallas import tpu_sc as plsc

N = 10000
D_IN = 256
Z_DIM = 256
H = 128
NP = 10240
E = 320000
CW = 128
NS = 16
NC = 2
CH = 160
EPAD = NS * CH * CW
CWD = 80
CHD = 256
EPADD = NS * CHD * CWD
ROWS_PER_TILE = NP // NS
BLK = 1024


def _mmscale_body(x_ref, w_ref, b_ref, dp_ref, g_ref, hi_ref):
    j = pl.program_id(1)

    @pl.when(j == 0)
    def _():
        h = lax.dot_general(x_ref[...], w_ref[...],
                            dimension_numbers=(((1,), (1,)), ((), ())),
                            preferred_element_type=jnp.float32)
        h = h + b_ref[...]
        l2 = jnp.sqrt(jnp.sum(h * h, axis=1, keepdims=True))
        h = h / jnp.maximum(l2, 1e-12) * 1.8
        cnt = dp_ref[0] + dp_ref[1] + 1.0
        g = h * lax.rsqrt(cnt)
        g_ref[...] = g[:, :H]
        hi_ref[...] = g[:, H:]

    @pl.when(j == 1)
    def _():
        g_ref[...] = hi_ref[...]


def _mmscale(x, W, b, dparts):
    return pl.pallas_call(
        _mmscale_body,
        grid=(NP // BLK, 2),
        in_specs=[
            pl.BlockSpec((BLK, D_IN), lambda i, j: (i, 0)),
            pl.BlockSpec((Z_DIM, D_IN), lambda i, j: (0, 0)),
            pl.BlockSpec((1, Z_DIM), lambda i, j: (0, 0)),
            pl.BlockSpec((2, BLK, 1), lambda i, j: (0, i, 0)),
        ],
        out_specs=pl.BlockSpec((BLK, H), lambda i, j: (i + (NP // BLK) * j, 0)),
        out_shape=jax.ShapeDtypeStruct((2 * NP, H), jnp.float32),
        scratch_shapes=[pltpu.VMEM((BLK, H), jnp.float32)],
    )(x, W, b.reshape(1, Z_DIM), dparts)


def _deg_body(dsti, out, dstv, onesb, zb, dacc):
    c = lax.axis_index("c")
    s = lax.axis_index("s")
    pltpu.sync_copy(dsti.at[s, pl.ds((CH // 2) * c, CH // 2)], dstv)
    for j in range(8):
        onesb[pl.ds(16 * j, 16)] = jnp.ones((16,), jnp.float32)

    @pl.loop(0, ROWS_PER_TILE, step=16)
    def _(r):
        zb[pl.ds(r, 16)] = jnp.zeros((16,), jnp.float32)

    pltpu.sync_copy(zb, dacc.at[pl.ds(s * ROWS_PER_TILE, ROWS_PER_TILE)])
    plsc.subcore_barrier()

    @pl.loop(0, CH // 2)
    def _(ch):
        pltpu.sync_copy(onesb, dacc.at[dstv.at[ch]], add=True)

    plsc.subcore_barrier()
    pltpu.sync_copy(dacc.at[pl.ds(s * ROWS_PER_TILE, ROWS_PER_TILE)],
                    out.at[pl.ds(c * NP + s * ROWS_PER_TILE, ROWS_PER_TILE)])


def _deg_counts(dsti):
    mesh = plsc.VectorSubcoreMesh(core_axis_name="c", subcore_axis_name="s")
    k = pl.kernel(
        _deg_body,
        out_type=jax.ShapeDtypeStruct((NC * NP,), jnp.float32),
        mesh=mesh,
        scratch_types=[
            pltpu.VMEM((CH // 2, CW), jnp.int32),
            pltpu.VMEM((CW,), jnp.float32),
            pltpu.VMEM((ROWS_PER_TILE,), jnp.float32),
            pltpu.VMEM_SHARED((NP,), jnp.float32),
        ],
    )
    return k(dsti)


SB = 32


def _prop_body(gflat, srci, dsti, out, srcv, dstv, bufa, bufb, bufc, bufd,
               accs, sema, semb, semc, semd):
    c = lax.axis_index("c")
    s = lax.axis_index("s")

    @pl.loop(0, CWD)
    def _(r):
        for j in range(8):
            bufa[r, pl.ds(j * 16, 16)] = jnp.zeros((16,), jnp.float32)

    @pl.loop(0, ROWS_PER_TILE // CWD)
    def _(i):
        pltpu.sync_copy(
            bufa, accs.at[pl.ds(s * ROWS_PER_TILE + i * CWD, CWD)])

    plsc.subcore_barrier()

    def wait(buf, sem):
        pltpu.make_async_copy(gflat.at[pl.ds(0, CWD)], buf, sem).wait()

    @pl.loop(0, CHD // SB)
    def _(sb):
        st = pl.multiple_of(sb * SB, SB)
        pltpu.sync_copy(srci.at[c, s, pl.ds(st, SB)], srcv)
        pltpu.sync_copy(dsti.at[s, pl.ds(st, SB)], dstv)

        pltpu.async_copy(gflat.at[srcv.at[0]], bufa, sema)
        pltpu.async_copy(gflat.at[srcv.at[1]], bufb, semb)
        pltpu.async_copy(gflat.at[srcv.at[2]], bufc, semc)

        @pl.loop(0, SB // 4)
        def _(i):
            j = i * 4
            wait(bufa, sema)
            pltpu.async_copy(gflat.at[srcv.at[j + 3]], bufd, semd)
            pltpu.sync_copy(bufa, accs.at[dstv.at[j]], add=True)
            wait(bufb, semb)

            @pl.when(j + 4 < SB)
            def _():
                pltpu.async_copy(gflat.at[srcv.at[j + 4]], bufa, sema)

            pltpu.sync_copy(bufb, accs.at[dstv.at[j + 1]], add=True)
            wait(bufc, semc)

            @pl.when(j + 5 < SB)
            def _():
                pltpu.async_copy(gflat.at[srcv.at[j + 5]], bufb, semb)

            pltpu.sync_copy(bufc, accs.at[dstv.at[j + 2]], add=True)
            wait(bufd, semd)

            @pl.when(j + 6 < SB)
            def _():
                pltpu.async_copy(gflat.at[srcv.at[j + 6]], bufc, semc)

            pltpu.sync_copy(bufd, accs.at[dstv.at[j + 3]], add=True)

    plsc.subcore_barrier()
    pltpu.sync_copy(accs.at[pl.ds(s * ROWS_PER_TILE, ROWS_PER_TILE)],
                    out.at[pl.ds(c * NP + s * ROWS_PER_TILE, ROWS_PER_TILE)])


def _propagate(gflat, srci, dsti):
    mesh = plsc.VectorSubcoreMesh(core_axis_name="c", subcore_axis_name="s")
    k = pl.kernel(
        _prop_body,
        out_type=jax.ShapeDtypeStruct((NC * NP, H), jnp.float32),
        mesh=mesh,
        scratch_types=[
            pltpu.VMEM((SB, CWD), jnp.int32),
            pltpu.VMEM((SB, CWD), jnp.int32),
            pltpu.VMEM((CWD, H), jnp.float32),
            pltpu.VMEM((CWD, H), jnp.float32),
            pltpu.VMEM((CWD, H), jnp.float32),
            pltpu.VMEM((CWD, H), jnp.float32),
            pltpu.VMEM_SHARED((NP, H), jnp.float32),
            pltpu.SemaphoreType.DMA,
            pltpu.SemaphoreType.DMA,
            pltpu.SemaphoreType.DMA,
            pltpu.SemaphoreType.DMA,
        ],
    )
    return k(gflat, srci, dsti)


def _combine_body(acc_ref, g_ref, dp_ref, o_ref):
    cnt = dp_ref[0] + dp_ref[1] + 1.0
    dinv = lax.rsqrt(cnt)
    o_ref[...] = dinv * (acc_ref[...] + g_ref[...])


def _combine(accflat, gflat, dparts):
    return pl.pallas_call(
        _combine_body,
        grid=(NP // BLK, 2),
        in_specs=[
            pl.BlockSpec((BLK, H), lambda i, j: (i + (NP // BLK) * j, 0)),
            pl.BlockSpec((BLK, H), lambda i, j: (i + (NP // BLK) * j, 0)),
            pl.BlockSpec((2, BLK, 1), lambda i, j: (0, i, 0)),
        ],
        out_specs=pl.BlockSpec((BLK, H), lambda i, j: (i, j)),
        out_shape=jax.ShapeDtypeStruct((N, Z_DIM), jnp.float32),
    )(accflat, gflat, dparts)


def kernel(x, edge_index, W, b):
    src = edge_index[0].astype(jnp.int32)
    dst = edge_index[1].astype(jnp.int32)
    pad_b = jnp.arange(EPAD - E, dtype=jnp.int32)
    dsti = jnp.concatenate([dst, N + pad_b % 128]).reshape(NS, CH, CW)
    pad_d = jnp.arange(EPADD - E, dtype=jnp.int32)
    srci_h = jnp.concatenate([src, pad_d % N]).reshape(NS, CHD, CWD)
    srci_d = jnp.stack([srci_h, srci_h + NP])
    dsti_d = jnp.concatenate([dst, N + pad_d % 128]).reshape(NS, CHD, CWD)

    dcounts = _deg_counts(dsti)
    dparts = dcounts.reshape(2, NP, 1)
    gflat = _mmscale(x, W, b, dparts)
    accflat = _propagate(gflat, srci_d, dsti_d)
    return _combine(accflat, gflat, dparts)


# speedup vs baseline: 1.3688x; 1.0154x over previous
import functools

import jax
import jax.numpy as jnp
from jax import lax
from jax.experimental import pallas as pl
from jax.experimental.pallas import tpu as pltpu
from jax.experimental.pallas import tpu_sc as plsc

N = 10000
D_IN = 256
Z_DIM = 256
H = 128
NP = 10240
E = 320000
CW = 128
NS = 16
NC = 2
CH = 160
EPAD = NS * CH * CW
CWD = 80
CHD = 256
EPADD = NS * CHD * CWD
ROWS_PER_TILE = NP // NS
BLK = 1024


def _mmscale_body(x_ref, w_ref, b_ref, dp_ref, g_ref, hi_ref):
    j = pl.program_id(1)

    @pl.when(j == 0)
    def _():
        h = lax.dot_general(x_ref[...], w_ref[...],
                            dimension_numbers=(((1,), (1,)), ((), ())),
                            preferred_element_type=jnp.float32)
        h = h + b_ref[...]
        l2 = jnp.sqrt(jnp.sum(h * h, axis=1, keepdims=True))
        h = h / jnp.maximum(l2, 1e-12) * 1.8
        cnt = dp_ref[0] + dp_ref[1] + 1.0
        g = h * lax.rsqrt(cnt)
        g_ref[...] = g[:, :H]
        hi_ref[...] = g[:, H:]

    @pl.when(j == 1)
    def _():
        g_ref[...] = hi_ref[...]


def _mmscale(x, W, b, dparts):
    return pl.pallas_call(
        _mmscale_body,
        grid=(NP // BLK, 2),
        in_specs=[
            pl.BlockSpec((BLK, D_IN), lambda i, j: (i, 0)),
            pl.BlockSpec((Z_DIM, D_IN), lambda i, j: (0, 0)),
            pl.BlockSpec((1, Z_DIM), lambda i, j: (0, 0)),
            pl.BlockSpec((2, BLK, 1), lambda i, j: (0, i, 0)),
        ],
        out_specs=pl.BlockSpec((BLK, H), lambda i, j: (i + (NP // BLK) * j, 0)),
        out_shape=jax.ShapeDtypeStruct((2 * NP, H), jnp.float32),
        scratch_shapes=[pltpu.VMEM((BLK, H), jnp.float32)],
    )(x, W, b.reshape(1, Z_DIM), dparts)


def _deg_body(dsti, out, dstv, onesb, zb, dacc, semo):
    c = lax.axis_index("c")
    s = lax.axis_index("s")
    pltpu.sync_copy(dsti.at[s, pl.ds((CH // 2) * c, CH // 2)], dstv)
    for j in range(8):
        onesb[pl.ds(16 * j, 16)] = jnp.ones((16,), jnp.float32)

    @pl.loop(0, ROWS_PER_TILE, step=16)
    def _(r):
        zb[pl.ds(r, 16)] = jnp.zeros((16,), jnp.float32)

    pltpu.sync_copy(zb, dacc.at[pl.ds(s * ROWS_PER_TILE, ROWS_PER_TILE)])
    plsc.subcore_barrier()

    @pl.loop(0, CH // 2)
    def _(ch):
        pltpu.async_copy(onesb, dacc.at[dstv.at[ch]], semo, add=True)

    @pl.loop(0, CH // 2)
    def _(ch):
        pltpu.make_async_copy(onesb, dacc.at[pl.ds(0, CW)], semo).wait()

    plsc.subcore_barrier()
    pltpu.sync_copy(dacc.at[pl.ds(s * ROWS_PER_TILE, ROWS_PER_TILE)],
                    out.at[pl.ds(c * NP + s * ROWS_PER_TILE, ROWS_PER_TILE)])


def _deg_counts(dsti):
    mesh = plsc.VectorSubcoreMesh(core_axis_name="c", subcore_axis_name="s")
    k = pl.kernel(
        _deg_body,
        out_type=jax.ShapeDtypeStruct((NC * NP,), jnp.float32),
        mesh=mesh,
        scratch_types=[
            pltpu.VMEM((CH // 2, CW), jnp.int32),
            pltpu.VMEM((CW,), jnp.float32),
            pltpu.VMEM((ROWS_PER_TILE,), jnp.float32),
            pltpu.VMEM_SHARED((NP,), jnp.float32),
            pltpu.SemaphoreType.DMA,
        ],
    )
    return k(dsti)


SB = 32


def _prop_body(gflat, srci, dsti, out, srcv, dstv, bufa, bufb, bufc, bufd,
               accs, sema, semb, semc, semd):
    c = lax.axis_index("c")
    s = lax.axis_index("s")

    @pl.loop(0, CWD)
    def _(r):
        for j in range(8):
            bufa[r, pl.ds(j * 16, 16)] = jnp.zeros((16,), jnp.float32)

    @pl.loop(0, ROWS_PER_TILE // CWD)
    def _(i):
        pltpu.sync_copy(
            bufa, accs.at[pl.ds(s * ROWS_PER_TILE + i * CWD, CWD)])

    plsc.subcore_barrier()

    def wait(buf, sem):
        pltpu.make_async_copy(gflat.at[pl.ds(0, CWD)], buf, sem).wait()

    @pl.loop(0, CHD // SB)
    def _(sb):
        st = pl.multiple_of(sb * SB, SB)
        pltpu.sync_copy(srci.at[c, s, pl.ds(st, SB)], srcv)
        pltpu.sync_copy(dsti.at[s, pl.ds(st, SB)], dstv)

        pltpu.async_copy(gflat.at[srcv.at[0]], bufa, sema)
        pltpu.async_copy(gflat.at[srcv.at[1]], bufb, semb)
        pltpu.async_copy(gflat.at[srcv.at[2]], bufc, semc)

        @pl.loop(0, SB // 4)
        def _(i):
            j = i * 4
            wait(bufa, sema)
            pltpu.async_copy(gflat.at[srcv.at[j + 3]], bufd, semd)
            pltpu.sync_copy(bufa, accs.at[dstv.at[j]], add=True)
            wait(bufb, semb)

            @pl.when(j + 4 < SB)
            def _():
                pltpu.async_copy(gflat.at[srcv.at[j + 4]], bufa, sema)

            pltpu.sync_copy(bufb, accs.at[dstv.at[j + 1]], add=True)
            wait(bufc, semc)

            @pl.when(j + 5 < SB)
            def _():
                pltpu.async_copy(gflat.at[srcv.at[j + 5]], bufb, semb)

            pltpu.sync_copy(bufc, accs.at[dstv.at[j + 2]], add=True)
            wait(bufd, semd)

            @pl.when(j + 6 < SB)
            def _():
                pltpu.async_copy(gflat.at[srcv.at[j + 6]], bufc, semc)

            pltpu.sync_copy(bufd, accs.at[dstv.at[j + 3]], add=True)

    plsc.subcore_barrier()
    pltpu.sync_copy(accs.at[pl.ds(s * ROWS_PER_TILE, ROWS_PER_TILE)],
                    out.at[pl.ds(c * NP + s * ROWS_PER_TILE, ROWS_PER_TILE)])


def _propagate(gflat, srci, dsti):
    mesh = plsc.VectorSubcoreMesh(core_axis_name="c", subcore_axis_name="s")
    k = pl.kernel(
        _prop_body,
        out_type=jax.ShapeDtypeStruct((NC * NP, H), jnp.float32),
        mesh=mesh,
        scratch_types=[
            pltpu.VMEM((SB, CWD), jnp.int32),
            pltpu.VMEM((SB, CWD), jnp.int32),
            pltpu.VMEM((CWD, H), jnp.float32),
            pltpu.VMEM((CWD, H), jnp.float32),
            pltpu.VMEM((CWD, H), jnp.float32),
            pltpu.VMEM((CWD, H), jnp.float32),
            pltpu.VMEM_SHARED((NP, H), jnp.float32),
            pltpu.SemaphoreType.DMA,
            pltpu.SemaphoreType.DMA,
            pltpu.SemaphoreType.DMA,
            pltpu.SemaphoreType.DMA,
        ],
    )
    return k(gflat, srci, dsti)


def _combine_body(acc_ref, g_ref, dp_ref, o_ref):
    cnt = dp_ref[0] + dp_ref[1] + 1.0
    dinv = lax.rsqrt(cnt)
    o_ref[...] = dinv * (acc_ref[...] + g_ref[...])


def _combine(accflat, gflat, dparts):
    return pl.pallas_call(
        _combine_body,
        grid=(NP // BLK, 2),
        in_specs=[
            pl.BlockSpec((BLK, H), lambda i, j: (i + (NP // BLK) * j, 0)),
            pl.BlockSpec((BLK, H), lambda i, j: (i + (NP // BLK) * j, 0)),
            pl.BlockSpec((2, BLK, 1), lambda i, j: (0, i, 0)),
        ],
        out_specs=pl.BlockSpec((BLK, H), lambda i, j: (i, j)),
        out_shape=jax.ShapeDtypeStruct((N, Z_DIM), jnp.float32),
    )(accflat, gflat, dparts)


def kernel(x, edge_index, W, b):
    src = edge_index[0].astype(jnp.int32)
    dst = edge_index[1].astype(jnp.int32)
    pad_b = jnp.arange(EPAD - E, dtype=jnp.int32)
    dsti = jnp.concatenate([dst, N + pad_b % 128]).reshape(NS, CH, CW)
    pad_d = jnp.arange(EPADD - E, dtype=jnp.int32)
    srci_h = jnp.concatenate([src, pad_d % N]).reshape(NS, CHD, CWD)
    srci_d = jnp.stack([srci_h, srci_h + NP])
    dsti_d = jnp.concatenate([dst, N + pad_d % 128]).reshape(NS, CHD, CWD)

    dcounts = _deg_counts(dsti)
    dparts = dcounts.reshape(2, NP, 1)
    gflat = _mmscale(x, W, b, dparts)
    accflat = _propagate(gflat, srci_d, dsti_d)
    return _combine(accflat, gflat, dparts)
